# Initial kernel scaffold; baseline (speedup 1.0000x reference)
import jax
import jax.numpy as jnp
from jax.experimental import pallas as pl


def kernel(x, edge_index, W1, g1, b1, W2, g2, b2, W3, g3, b3, fc_w, fc_b):
    raise NotImplementedError("write your pallas kernel here")


# speedup vs baseline: 13.1518x; 13.1518x over previous
import functools
import math

import jax
import jax.numpy as jnp
from jax import lax
from jax.experimental import pallas as pl
from jax.experimental.pallas import tpu as pltpu
from jax.experimental.pallas import tpu_sc as plsc

N = 10000
NP = 10240
D = 128
E = 320000
EPS = 1e-5

NC = 2
NS = 16
NW = NC * NS
CHUNK = 128
NCHUNKS = E // CHUNK
RPT = NP // NS
ZROWS = 128
DEGW = 16

_MESH = plsc.VectorSubcoreMesh(core_axis_name="c", subcore_axis_name="s")


def _nchunks_for(w):
    return (NCHUNKS - 1 - w) // NW + 1


def _deg_body(rows_hbm, zeros_hbm, ones_hbm, out_hbm, idx_v, ones_v, zbuf_v,
              acc_sh, sem):
    c = lax.axis_index("c")
    s = lax.axis_index("s")
    w = s * NC + c
    pltpu.sync_copy(zeros_hbm, zbuf_v)
    for j in range(RPT // ZROWS):
        pltpu.sync_copy(zbuf_v, acc_sh.at[pl.ds(s * RPT + j * ZROWS, ZROWS)])
    pltpu.sync_copy(ones_hbm, ones_v)
    plsc.subcore_barrier()

    def body(k, carry):
        base = (w + k * NW) * CHUNK
        pltpu.sync_copy(rows_hbm.at[pl.ds(base, CHUNK)], idx_v)
        pltpu.sync_copy(ones_v, acc_sh.at[idx_v], add=True)
        return carry

    lax.fori_loop(0, _nchunks_for(w), body, 0)
    plsc.subcore_barrier()
    for j in range(RPT // ZROWS):
        pltpu.sync_copy(acc_sh.at[pl.ds(s * RPT + j * ZROWS, ZROWS)], zbuf_v)
        pltpu.sync_copy(zbuf_v, out_hbm.at[c, pl.ds(s * RPT + j * ZROWS, ZROWS)])


_deg_call = pl.kernel(
    _deg_body,
    out_type=jax.ShapeDtypeStruct((NC, NP, D), jnp.float32),
    mesh=_MESH,
    scratch_types=[
        pltpu.VMEM((CHUNK,), jnp.int32),
        pltpu.VMEM((CHUNK, D), jnp.float32),
        pltpu.VMEM((ZROWS, D), jnp.float32),
        pltpu.VMEM_SHARED((NP, D), jnp.float32),
        pltpu.SemaphoreType.DMA,
    ],
)


def _scat_body(y_hbm, col_hbm, row_hbm, zeros_hbm, out_hbm,
               idxc_v, idxr_v, rows_v, zbuf_v, acc_sh, sem):
    c = lax.axis_index("c")
    s = lax.axis_index("s")
    w = s * NC + c
    pltpu.sync_copy(zeros_hbm, zbuf_v)
    for j in range(RPT // ZROWS):
        pltpu.sync_copy(zbuf_v, acc_sh.at[pl.ds(s * RPT + j * ZROWS, ZROWS)])
    plsc.subcore_barrier()

    def body(k, carry):
        base = (w + k * NW) * CHUNK
        pltpu.sync_copy(col_hbm.at[pl.ds(base, CHUNK)], idxc_v)
        pltpu.async_copy(y_hbm.at[idxc_v], rows_v, sem).wait()
        pltpu.sync_copy(row_hbm.at[pl.ds(base, CHUNK)], idxr_v)
        pltpu.sync_copy(rows_v, acc_sh.at[idxr_v], add=True)
        return carry

    lax.fori_loop(0, _nchunks_for(w), body, 0)
    plsc.subcore_barrier()
    for j in range(RPT // ZROWS):
        pltpu.sync_copy(acc_sh.at[pl.ds(s * RPT + j * ZROWS, ZROWS)], zbuf_v)
        pltpu.sync_copy(zbuf_v, out_hbm.at[c, pl.ds(s * RPT + j * ZROWS, ZROWS)])


_scat_call = pl.kernel(
    _scat_body,
    out_type=jax.ShapeDtypeStruct((NC, NP, D), jnp.float32),
    mesh=_MESH,
    scratch_types=[
        pltpu.VMEM((CHUNK,), jnp.int32),
        pltpu.VMEM((CHUNK,), jnp.int32),
        pltpu.VMEM((CHUNK, D), jnp.float32),
        pltpu.VMEM((ZROWS, D), jnp.float32),
        pltpu.VMEM_SHARED((NP, D), jnp.float32),
        pltpu.SemaphoreType.DMA,
    ],
)


RB = 1000


def _tc_first_body(x_ref, w_ref, d_ref, o_ref):
    o_ref[...] = (
        jnp.dot(x_ref[...], w_ref[...], preferred_element_type=jnp.float32)
        * d_ref[...]
    )


def _tc_mid_body(p_ref, y_ref, d_ref, s_ref, b_ref, w_ref, o_ref):
    t = (p_ref[0] + p_ref[1] + y_ref[...]) * d_ref[...]
    h = jnp.maximum(t * s_ref[...] + b_ref[...], 0.0)
    o_ref[...] = (
        jnp.dot(h, w_ref[...], preferred_element_type=jnp.float32) * d_ref[...]
    )


def _tc_final_body(p_ref, y_ref, d_ref, s_ref, b_ref, w_ref, fb_ref, o_ref):
    t = (p_ref[0] + p_ref[1] + y_ref[...]) * d_ref[...]
    h = jnp.maximum(t * s_ref[...] + b_ref[...], 0.0)
    o_ref[...] = (
        jnp.dot(h, w_ref[...], preferred_element_type=jnp.float32) + fb_ref[...]
    )


_spec_rows = pl.BlockSpec((RB, D), lambda i: (i, 0))
_spec_p = pl.BlockSpec((NC, RB, D), lambda i: (0, i, 0))
_spec_w = pl.BlockSpec((D, D), lambda i: (0, 0))
_spec_d = pl.BlockSpec((RB, 1), lambda i: (i, 0))
_spec_vec = pl.BlockSpec((1, D), lambda i: (0, 0))

_tc_first = pl.pallas_call(
    _tc_first_body,
    out_shape=jax.ShapeDtypeStruct((N, D), jnp.float32),
    grid=(N // RB,),
    in_specs=[_spec_rows, _spec_w, _spec_d],
    out_specs=_spec_rows,
)

_tc_mid = pl.pallas_call(
    _tc_mid_body,
    out_shape=jax.ShapeDtypeStruct((N, D), jnp.float32),
    grid=(N // RB,),
    in_specs=[_spec_p, _spec_rows, _spec_d, _spec_vec, _spec_vec, _spec_w],
    out_specs=_spec_rows,
)

_tc_final = pl.pallas_call(
    _tc_final_body,
    out_shape=jax.ShapeDtypeStruct((N, D), jnp.float32),
    grid=(N // RB,),
    in_specs=[_spec_p, _spec_rows, _spec_d, _spec_vec, _spec_vec, _spec_w,
              _spec_vec],
    out_specs=_spec_rows,
)


def kernel(x, edge_index, W1, g1, b1, W2, g2, b2, W3, g3, b3, fc_w, fc_b):
    row = edge_index[0].astype(jnp.int32)
    col = edge_index[1].astype(jnp.int32)

    ones_deg = jnp.ones((CHUNK, D), jnp.float32)
    zeros_d = jnp.zeros((ZROWS, D), jnp.float32)

    degp = _deg_call(row, zeros_d, ones_deg)
    deg = 1.0 + degp[0, :N, 0] + degp[1, :N, 0]
    dvec = lax.rsqrt(deg).reshape(N, 1)

    inv = 1.0 / math.sqrt(1.0 + EPS)
    s1 = (g1 * inv).reshape(1, D)
    s2 = (g2 * inv).reshape(1, D)
    s3 = (g3 * inv).reshape(1, D)
    b1r = b1.reshape(1, D)
    b2r = b2.reshape(1, D)
    b3r = b3.reshape(1, D)

    y1 = _tc_first(x, W1, dvec)
    p1 = _scat_call(y1, col, row, zeros_d)
    y2 = _tc_mid(p1, y1, dvec, s1, b1r, W2)
    p2 = _scat_call(y2, col, row, zeros_d)
    y3 = _tc_mid(p2, y2, dvec, s2, b2r, W3)
    p3 = _scat_call(y3, col, row, zeros_d)
    out = _tc_final(p3, y3, dvec, s3, b3r, fc_w.T, fc_b.reshape(1, D))
    return out
